# Initial kernel scaffold; baseline (speedup 1.0000x reference)
import functools
import jax
import jax.numpy as jnp
from jax import lax
from jax.experimental import pallas as pl
from jax.experimental.pallas import tpu as pltpu
from jax.experimental.pallas import tpu_sc as plsc

N = 10000
E = 320000
D = 128

NC = 2
NS = 16
NW = NC * NS
EPT = E // NW
K = 80
NCH = EPT // K
RPT = N // NS
RCH = 125

_MESH = plsc.VectorSubcoreMesh(core_axis_name="c", subcore_axis_name="s")


@functools.partial(
    pl.kernel,
    out_type=jax.ShapeDtypeStruct((NW, N), jnp.float32),
    mesh=_MESH,
    scratch_types=[
        pltpu.VMEM((EPT,), jnp.int32),
        pltpu.VMEM((N,), jnp.float32),
    ],
)
def _deg_kernel(ei_hbm, deg_out, idx_v, deg_v):
    c = lax.axis_index("c")
    s = lax.axis_index("s")
    wid = c * NS + s
    base = wid * EPT
    pltpu.sync_copy(ei_hbm.at[1, pl.ds(base, EPT)], idx_v)

    zeros = jnp.zeros((16,), jnp.float32)
    ones = jnp.ones((16,), jnp.float32)

    def zbody(i, carry):
        deg_v[pl.ds(i * 16, 16)] = zeros
        return carry

    lax.fori_loop(0, N // 16, zbody, 0)

    def sbody(i, carry):
        idx = idx_v[pl.ds(i * 16, 16)]
        plsc.addupdate_scatter(deg_v, [idx], ones)
        return carry

    lax.fori_loop(0, EPT // 16, sbody, 0)
    pltpu.sync_copy(deg_v, deg_out.at[wid])


@functools.partial(
    pl.kernel,
    out_type=jax.ShapeDtypeStruct((NC, N, D), jnp.float32),
    mesh=_MESH,
    scratch_types=[
        pltpu.VMEM((K,), jnp.int32),
        pltpu.VMEM((K,), jnp.int32),
        pltpu.VMEM((K, D), jnp.float32),
        pltpu.VMEM((RCH, D), jnp.float32),
        pltpu.VMEM_SHARED((N, D), jnp.float32),
        pltpu.SemaphoreType.DMA,
    ],
)
def _agg_kernel(hs_hbm, ei_hbm, part_out, src_v, dst_v, rows_v, wbuf, acc_sh, sem):
    c = lax.axis_index("c")
    s = lax.axis_index("s")
    wid = c * NS + s
    base = wid * EPT

    zeros = jnp.zeros((16,), jnp.float32)

    def zbody(t, carry):
        wbuf[t // (D // 16), pl.ds((t % (D // 16)) * 16, 16)] = zeros
        return carry

    lax.fori_loop(0, RCH * (D // 16), zbody, 0)
    for i in range(RPT // RCH):
        pltpu.sync_copy(wbuf, acc_sh.at[pl.ds(s * RPT + i * RCH, RCH), :])
    plsc.subcore_barrier()

    def body(i, carry):
        eb = base + i * K
        pltpu.sync_copy(ei_hbm.at[0, pl.ds(eb, K)], src_v)
        pltpu.sync_copy(ei_hbm.at[1, pl.ds(eb, K)], dst_v)
        pltpu.async_copy(hs_hbm.at[src_v], rows_v, sem).wait()
        pltpu.sync_copy(rows_v, acc_sh.at[dst_v], add=True)
        return carry

    lax.fori_loop(0, NCH, body, 0)
    plsc.subcore_barrier()

    for i in range(RPT // RCH):
        r0 = s * RPT + i * RCH
        pltpu.sync_copy(acc_sh.at[pl.ds(r0, RCH), :], wbuf)
        pltpu.sync_copy(wbuf, part_out.at[c, pl.ds(r0, RCH), :])


def _mm_body(x_ref, w_ref, degp_ref, hs_ref):
    deg = jnp.sum(degp_ref[...], axis=0) + 1.0
    dinv = lax.rsqrt(deg)
    h = jnp.dot(x_ref[...], w_ref[...], preferred_element_type=jnp.float32)
    hs_ref[...] = h * dinv[:, None]


def _out_body(p_ref, hs_ref, degp_ref, b_ref, o_ref):
    deg = jnp.sum(degp_ref[...], axis=0) + 1.0
    dinv = lax.rsqrt(deg)
    v = (p_ref[0] + p_ref[1] + hs_ref[...]) * dinv[:, None] + b_ref[...]
    m = jnp.max(v, axis=1, keepdims=True)
    z = v - m
    o_ref[...] = z - jnp.log(jnp.sum(jnp.exp(z), axis=1, keepdims=True))


_BN = 1000


def kernel(x, edge_index, W, b):
    degp = _deg_kernel(edge_index)

    hs = pl.pallas_call(
        _mm_body,
        grid=(N // _BN,),
        in_specs=[
            pl.BlockSpec((_BN, D), lambda i: (i, 0)),
            pl.BlockSpec((D, D), lambda i: (0, 0)),
            pl.BlockSpec((NW, _BN), lambda i: (0, i)),
        ],
        out_specs=pl.BlockSpec((_BN, D), lambda i: (i, 0)),
        out_shape=jax.ShapeDtypeStruct((N, D), jnp.float32),
    )(x, W, degp)

    parts = _agg_kernel(hs, edge_index)

    out = pl.pallas_call(
        _out_body,
        grid=(N // _BN,),
        in_specs=[
            pl.BlockSpec((NC, _BN, D), lambda i: (0, i, 0)),
            pl.BlockSpec((_BN, D), lambda i: (i, 0)),
            pl.BlockSpec((NW, _BN), lambda i: (0, i)),
            pl.BlockSpec((1, D), lambda i: (0, 0)),
        ],
        out_specs=pl.BlockSpec((_BN, D), lambda i: (i, 0)),
        out_shape=jax.ShapeDtypeStruct((N, D), jnp.float32),
    )(parts, hs, degp, b.reshape(1, D))

    return out


# speedup vs baseline: 20.8877x; 20.8877x over previous
import functools
import jax
import jax.numpy as jnp
from jax import lax
from jax.experimental import pallas as pl
from jax.experimental.pallas import tpu as pltpu
from jax.experimental.pallas import tpu_sc as plsc

N = 10000
NPAD = 10240
E = 320000
D = 128

NC = 2
NS = 16
NW = NC * NS
EPT = E // NW
K = 80
NCH = EPT // K
RPT = NPAD // NS
RCH = 128

_MESH = plsc.VectorSubcoreMesh(core_axis_name="c", subcore_axis_name="s")
_SC_PARAMS = pltpu.CompilerParams(needs_layout_passes=False)


@functools.partial(
    pl.kernel,
    out_type=jax.ShapeDtypeStruct((NW * NPAD,), jnp.float32),
    mesh=_MESH,
    scratch_types=[
        pltpu.VMEM((EPT,), jnp.int32),
        pltpu.VMEM((NPAD,), jnp.float32),
    ],
    compiler_params=_SC_PARAMS,
)
def _deg_kernel(dst_hbm, deg_out, idx_v, deg_v):
    c = lax.axis_index("c")
    s = lax.axis_index("s")
    wid = c * NS + s
    base = wid * EPT
    pltpu.sync_copy(dst_hbm.at[pl.ds(base, EPT)], idx_v)

    zeros = jnp.zeros((16,), jnp.float32)
    ones = jnp.ones((16,), jnp.float32)

    def zbody(i, carry):
        deg_v[pl.ds(i * 16, 16)] = zeros
        return carry

    lax.fori_loop(0, NPAD // 16, zbody, 0)

    def sbody(i, carry):
        idx = idx_v[pl.ds(i * 16, 16)]
        plsc.addupdate_scatter(deg_v, [idx], ones)
        return carry

    lax.fori_loop(0, EPT // 16, sbody, 0)
    pltpu.sync_copy(deg_v, deg_out.at[pl.ds(wid * NPAD, NPAD)])


@functools.partial(
    pl.kernel,
    out_type=jax.ShapeDtypeStruct((NC, NPAD, D), jnp.float32),
    mesh=_MESH,
    scratch_types=[
        pltpu.VMEM((K,), jnp.int32),
        pltpu.VMEM((K,), jnp.int32),
        pltpu.VMEM((K, D), jnp.float32),
        pltpu.VMEM((RCH, D), jnp.float32),
        pltpu.VMEM_SHARED((NPAD, D), jnp.float32),
        pltpu.SemaphoreType.DMA,
    ],
    compiler_params=_SC_PARAMS,
)
def _agg_kernel(hs_hbm, src_hbm, dst_hbm, part_out, src_v, dst_v, rows_v, wbuf,
                acc_sh, sem):
    c = lax.axis_index("c")
    s = lax.axis_index("s")
    wid = c * NS + s
    base = wid * EPT

    zeros = jnp.zeros((16,), jnp.float32)

    def zbody(t, carry):
        wbuf[t // (D // 16), pl.ds((t % (D // 16)) * 16, 16)] = zeros
        return carry

    lax.fori_loop(0, RCH * (D // 16), zbody, 0)
    for i in range(RPT // RCH):
        pltpu.sync_copy(wbuf, acc_sh.at[pl.ds(s * RPT + i * RCH, RCH), :])
    plsc.subcore_barrier()

    def body(i, carry):
        eb = base + i * K
        pltpu.sync_copy(src_hbm.at[pl.ds(eb, K)], src_v)
        pltpu.sync_copy(dst_hbm.at[pl.ds(eb, K)], dst_v)
        pltpu.async_copy(hs_hbm.at[src_v], rows_v, sem).wait()
        pltpu.sync_copy(rows_v, acc_sh.at[dst_v], add=True)
        return carry

    lax.fori_loop(0, NCH, body, 0)
    plsc.subcore_barrier()

    for i in range(RPT // RCH):
        r0 = s * RPT + i * RCH
        pltpu.sync_copy(acc_sh.at[pl.ds(r0, RCH), :], wbuf)
        pltpu.sync_copy(wbuf, part_out.at[c, pl.ds(r0, RCH), :])


def _mm_body(x_ref, w_ref, degp_ref, hs_ref):
    deg = jnp.sum(degp_ref[...], axis=1) + 1.0
    dinv = lax.rsqrt(deg)
    h = jnp.dot(x_ref[...], w_ref[...], preferred_element_type=jnp.float32)
    hs_ref[...] = h * dinv[:, None]


def _out_body(p_ref, hs_ref, degp_ref, b_ref, o_ref):
    deg = jnp.sum(degp_ref[...], axis=1) + 1.0
    dinv = lax.rsqrt(deg)
    v = (p_ref[0] + p_ref[1] + hs_ref[...]) * dinv[:, None] + b_ref[...]
    m = jnp.max(v, axis=1, keepdims=True)
    z = v - m
    o_ref[...] = z - jnp.log(jnp.sum(jnp.exp(z), axis=1, keepdims=True))


_BN = 1000


def kernel(x, edge_index, W, b):
    src = edge_index[0]
    dst = edge_index[1]
    degp = _deg_kernel(dst).reshape(NW, NPAD).T

    hs = pl.pallas_call(
        _mm_body,
        grid=(N // _BN,),
        in_specs=[
            pl.BlockSpec((_BN, D), lambda i: (i, 0)),
            pl.BlockSpec((D, D), lambda i: (0, 0)),
            pl.BlockSpec((_BN, NW), lambda i: (i, 0)),
        ],
        out_specs=pl.BlockSpec((_BN, D), lambda i: (i, 0)),
        out_shape=jax.ShapeDtypeStruct((N, D), jnp.float32),
    )(x, W, degp)

    parts = _agg_kernel(hs, src, dst)

    out = pl.pallas_call(
        _out_body,
        grid=(N // _BN,),
        in_specs=[
            pl.BlockSpec((NC, _BN, D), lambda i: (0, i, 0)),
            pl.BlockSpec((_BN, D), lambda i: (i, 0)),
            pl.BlockSpec((_BN, NW), lambda i: (i, 0)),
            pl.BlockSpec((1, D), lambda i: (0, 0)),
        ],
        out_specs=pl.BlockSpec((_BN, D), lambda i: (i, 0)),
        out_shape=jax.ShapeDtypeStruct((N, D), jnp.float32),
    )(parts, hs, degp, b.reshape(1, D))

    return out


# speedup vs baseline: 44.4155x; 2.1264x over previous
import functools
import jax
import jax.numpy as jnp
from jax import lax
from jax.experimental import pallas as pl
from jax.experimental.pallas import tpu as pltpu
from jax.experimental.pallas import tpu_sc as plsc

N = 10000
NPAD = 10240
E = 320000
D = 128

NC = 2
NS = 16
NW = NC * NS
EPT = E // NW
K = 128
NCH = 80
EPTP = NCH * K
EPAD = NW * EPTP
NB = 2
RPT = NPAD // NS
RCH = 128

_MESH = plsc.VectorSubcoreMesh(core_axis_name="c", subcore_axis_name="s")
_SC_PARAMS = pltpu.CompilerParams(needs_layout_passes=False)


@functools.partial(
    pl.kernel,
    out_type=jax.ShapeDtypeStruct((NW * NPAD,), jnp.float32),
    mesh=_MESH,
    scratch_types=[
        pltpu.VMEM((EPT,), jnp.int32),
        pltpu.VMEM((NPAD,), jnp.float32),
    ],
    compiler_params=_SC_PARAMS,
)
def _deg_kernel(dst_hbm, deg_out, idx_v, deg_v):
    c = lax.axis_index("c")
    s = lax.axis_index("s")
    wid = c * NS + s
    base = wid * EPT
    pltpu.sync_copy(dst_hbm.at[pl.ds(base, EPT)], idx_v)

    zeros = jnp.zeros((16,), jnp.float32)
    ones = jnp.ones((16,), jnp.float32)

    def zbody(i, carry):
        deg_v[pl.ds(i * 16, 16)] = zeros
        return carry

    lax.fori_loop(0, NPAD // 16, zbody, 0)

    def sbody(i, carry):
        idx = idx_v[pl.ds(i * 16, 16)]
        plsc.addupdate_scatter(deg_v, [idx], ones)
        return carry

    lax.fori_loop(0, EPT // 16, sbody, 0)
    pltpu.sync_copy(deg_v, deg_out.at[pl.ds(wid * NPAD, NPAD)])


@functools.partial(
    pl.kernel,
    out_type=jax.ShapeDtypeStruct((NC, NPAD, D), jnp.float32),
    mesh=_MESH,
    scratch_types=[
        pltpu.VMEM((NCH, K), jnp.int32),
        [pltpu.VMEM((K,), jnp.int32)] * NB,
        [pltpu.VMEM((K, D), jnp.float32)] * NB,
        pltpu.VMEM_SHARED((NPAD, D), jnp.float32),
        [pltpu.SemaphoreType.DMA] * NB,
        [pltpu.SemaphoreType.DMA] * NB,
    ],
    compiler_params=_SC_PARAMS,
)
def _agg_kernel(hs_hbm, src_hbm, dst_hbm, part_out, sidx, didx, rows,
                acc_sh, sem_g, sem_d):
    c = lax.axis_index("c")
    s = lax.axis_index("s")
    wid = c * NS + s
    base = wid * EPTP

    pltpu.sync_copy(src_hbm.at[wid], sidx)

    zeros = jnp.zeros((16,), jnp.float32)

    def zbody(t, carry):
        rows[0][t // (D // 16), pl.ds((t % (D // 16)) * 16, 16)] = zeros
        return carry

    lax.fori_loop(0, RCH * (D // 16), zbody, 0)
    for i in range(RPT // RCH):
        pltpu.sync_copy(rows[0], acc_sh.at[pl.ds(s * RPT + i * RCH, RCH), :])
    plsc.subcore_barrier()

    def start(ch, b):
        pltpu.async_copy(dst_hbm.at[pl.ds(base + ch * K, K)], didx[b],
                         sem_d[b])
        pltpu.async_copy(hs_hbm.at[sidx.at[ch]], rows[b], sem_g[b])

    for b in range(NB):
        start(b, b)

    def body(g, carry):
        for b in range(NB):
            ch = g * NB + b
            pltpu.make_async_copy(hs_hbm.at[sidx.at[ch]], rows[b],
                                  sem_g[b]).wait()
            pltpu.make_async_copy(dst_hbm.at[pl.ds(base + ch * K, K)],
                                  didx[b], sem_d[b]).wait()
            pltpu.sync_copy(rows[b], acc_sh.at[didx[b]], add=True)

            @pl.when(ch + NB < NCH)
            def _():
                start(ch + NB, b)

        return carry

    lax.fori_loop(0, NCH // NB, body, 0)
    plsc.subcore_barrier()

    for i in range(RPT // RCH):
        r0 = s * RPT + i * RCH
        pltpu.sync_copy(acc_sh.at[pl.ds(r0, RCH), :], rows[0])
        pltpu.sync_copy(rows[0], part_out.at[c, pl.ds(r0, RCH), :])


def _mm_body(x_ref, w_ref, degp_ref, hs_ref):
    deg = jnp.sum(degp_ref[...], axis=1) + 1.0
    dinv = lax.rsqrt(deg)
    h = jnp.dot(x_ref[...], w_ref[...], preferred_element_type=jnp.float32)
    hs_ref[...] = h * dinv[:, None]


def _out_body(p_ref, hs_ref, degp_ref, b_ref, o_ref):
    deg = jnp.sum(degp_ref[...], axis=1) + 1.0
    dinv = lax.rsqrt(deg)
    v = (p_ref[0] + p_ref[1] + hs_ref[...]) * dinv[:, None] + b_ref[...]
    m = jnp.max(v, axis=1, keepdims=True)
    z = v - m
    o_ref[...] = z - jnp.log(jnp.sum(jnp.exp(z), axis=1, keepdims=True))


_BN = 1000


def kernel(x, edge_index, W, b):
    src = edge_index[0]
    dst = edge_index[1]
    pad = jnp.arange(EPAD - E, dtype=jnp.int32)
    srcp = jnp.concatenate([src, pad % N]).reshape(NW, NCH, K)
    dstp = jnp.concatenate([dst, N + pad % (NPAD - N)])
    degp = _deg_kernel(dst).reshape(NW, NPAD).T

    hs = pl.pallas_call(
        _mm_body,
        grid=(N // _BN,),
        in_specs=[
            pl.BlockSpec((_BN, D), lambda i: (i, 0)),
            pl.BlockSpec((D, D), lambda i: (0, 0)),
            pl.BlockSpec((_BN, NW), lambda i: (i, 0)),
        ],
        out_specs=pl.BlockSpec((_BN, D), lambda i: (i, 0)),
        out_shape=jax.ShapeDtypeStruct((N, D), jnp.float32),
    )(x, W, degp)

    parts = _agg_kernel(hs, srcp, dstp)

    out = pl.pallas_call(
        _out_body,
        grid=(N // _BN,),
        in_specs=[
            pl.BlockSpec((NC, _BN, D), lambda i: (0, i, 0)),
            pl.BlockSpec((_BN, D), lambda i: (i, 0)),
            pl.BlockSpec((_BN, NW), lambda i: (i, 0)),
            pl.BlockSpec((1, D), lambda i: (0, 0)),
        ],
        out_specs=pl.BlockSpec((_BN, D), lambda i: (i, 0)),
        out_shape=jax.ShapeDtypeStruct((N, D), jnp.float32),
    )(parts, hs, degp, b.reshape(1, D))

    return out


# speedup vs baseline: 46.7876x; 1.0534x over previous
import functools
import jax
import jax.numpy as jnp
from jax import lax
from jax.experimental import pallas as pl
from jax.experimental.pallas import tpu as pltpu
from jax.experimental.pallas import tpu_sc as plsc

N = 10000
NPAD = 10240
E = 320000
D = 128

NC = 2
NS = 16
NW = NC * NS
EPT = E // NW
K = 80
NCH = EPT // K
NSLOT = 4
RPT = NPAD // NS

_MESH = plsc.VectorSubcoreMesh(core_axis_name="c", subcore_axis_name="s")
_SC_PARAMS = pltpu.CompilerParams(needs_layout_passes=False)


@functools.partial(
    pl.kernel,
    out_type=jax.ShapeDtypeStruct((NW * NPAD,), jnp.float32),
    mesh=_MESH,
    scratch_types=[
        pltpu.VMEM((EPT,), jnp.int32),
        pltpu.VMEM((NPAD,), jnp.float32),
    ],
    compiler_params=_SC_PARAMS,
)
def _deg_kernel(dst_hbm, deg_out, idx_v, deg_v):
    c = lax.axis_index("c")
    s = lax.axis_index("s")
    wid = c * NS + s
    base = wid * EPT
    pltpu.sync_copy(dst_hbm.at[pl.ds(base, EPT)], idx_v)

    zeros = jnp.zeros((16,), jnp.float32)
    ones = jnp.ones((16,), jnp.float32)

    def zbody(i, carry):
        deg_v[pl.ds(i * 16, 16)] = zeros
        return carry

    lax.fori_loop(0, NPAD // 16, zbody, 0)

    def sbody(i, carry):
        idx = idx_v[pl.ds(i * 16, 16)]
        plsc.addupdate_scatter(deg_v, [idx], ones)
        return carry

    lax.fori_loop(0, EPT // 16, sbody, 0)
    pltpu.sync_copy(deg_v, deg_out.at[pl.ds(wid * NPAD, NPAD)])


@functools.partial(
    pl.kernel,
    out_type=jax.ShapeDtypeStruct((NC, NPAD, D), jnp.float32),
    mesh=_MESH,
    scratch_types=[
        [pltpu.VMEM((K,), jnp.int32)] * NSLOT,
        [pltpu.VMEM((K,), jnp.int32)] * NSLOT,
        [pltpu.VMEM((K, D), jnp.float32)] * NSLOT,
        pltpu.VMEM_SHARED((NPAD, D), jnp.float32),
        [pltpu.SemaphoreType.DMA] * NSLOT,
        [pltpu.SemaphoreType.DMA] * NSLOT,
        [pltpu.SemaphoreType.DMA] * NSLOT,
        [pltpu.SemaphoreType.DMA] * NSLOT,
    ],
    compiler_params=_SC_PARAMS,
)
def _agg_kernel(hs_hbm, src_hbm, dst_hbm, part_out, sidx, didx, rows,
                acc_sh, sem_si, sem_d, sem_g, sem_s):
    c = lax.axis_index("c")
    s = lax.axis_index("s")
    wid = c * NS + s
    base = wid * EPT

    zeros = jnp.zeros((16,), jnp.float32)

    def zbody(t, carry):
        rows[0][t // (D // 16), pl.ds((t % (D // 16)) * 16, 16)] = zeros
        return carry

    lax.fori_loop(0, K * (D // 16), zbody, 0)
    for i in range(RPT // K):
        pltpu.sync_copy(rows[0], acc_sh.at[pl.ds(s * RPT + i * K, K), :])
    plsc.subcore_barrier()

    def sidx_cp(t, b):
        return pltpu.make_async_copy(src_hbm.at[pl.ds(base + t * K, K)],
                                     sidx[b], sem_si[b])

    def didx_cp(t, b):
        return pltpu.make_async_copy(dst_hbm.at[pl.ds(base + t * K, K)],
                                     didx[b], sem_d[b])

    for t0 in range(2):
        sidx_cp(t0, t0).start()

    def body(g, carry):
        for b in range(NSLOT):
            t = g * NSLOT + b
            b2 = (b + 2) % NSLOT

            @pl.when(jnp.logical_and(t >= NSLOT, t < NCH + NSLOT))
            def _():
                pltpu.make_async_copy(rows[b], acc_sh.at[didx[b]],
                                      sem_s[b]).wait()

            @pl.when(t < NCH)
            def _():
                didx_cp(t, b).start()

            @pl.when(t < NCH)
            def _():
                sidx_cp(t, b).wait()
                pltpu.async_copy(hs_hbm.at[sidx[b]], rows[b], sem_g[b])

            @pl.when(jnp.logical_and(t >= 2, t < NCH + 2))
            def _():
                pltpu.make_async_copy(hs_hbm.at[sidx[b2]], rows[b2],
                                      sem_g[b2]).wait()
                didx_cp(t - 2, b2).wait()
                pltpu.async_copy(rows[b2], acc_sh.at[didx[b2]], sem_s[b2],
                                 add=True)

            @pl.when(t + 2 < NCH)
            def _():
                sidx_cp(t + 2, b2).start()

        return carry

    lax.fori_loop(0, (NCH + NSLOT + NSLOT - 1) // NSLOT, body, 0)
    plsc.subcore_barrier()

    for i in range(RPT // K):
        r0 = s * RPT + i * K
        pltpu.sync_copy(acc_sh.at[pl.ds(r0, K), :],
                        part_out.at[c, pl.ds(r0, K), :])


def _mm_body(x_ref, w_ref, degp_ref, hs_ref):
    deg = jnp.sum(degp_ref[...], axis=1) + 1.0
    dinv = lax.rsqrt(deg)
    h = jnp.dot(x_ref[...], w_ref[...], preferred_element_type=jnp.float32)
    hs_ref[...] = h * dinv[:, None]


def _out_body(p_ref, hs_ref, degp_ref, b_ref, o_ref):
    deg = jnp.sum(degp_ref[...], axis=1) + 1.0
    dinv = lax.rsqrt(deg)
    v = (p_ref[0] + p_ref[1] + hs_ref[...]) * dinv[:, None] + b_ref[...]
    m = jnp.max(v, axis=1, keepdims=True)
    z = v - m
    o_ref[...] = z - jnp.log(jnp.sum(jnp.exp(z), axis=1, keepdims=True))


_BN = 1000


def kernel(x, edge_index, W, b):
    src = edge_index[0]
    dst = edge_index[1]
    degp = _deg_kernel(dst).reshape(NW, NPAD).T

    hs = pl.pallas_call(
        _mm_body,
        grid=(N // _BN,),
        in_specs=[
            pl.BlockSpec((_BN, D), lambda i: (i, 0)),
            pl.BlockSpec((D, D), lambda i: (0, 0)),
            pl.BlockSpec((_BN, NW), lambda i: (i, 0)),
        ],
        out_specs=pl.BlockSpec((_BN, D), lambda i: (i, 0)),
        out_shape=jax.ShapeDtypeStruct((N, D), jnp.float32),
    )(x, W, degp)

    parts = _agg_kernel(hs, src, dst)

    out = pl.pallas_call(
        _out_body,
        grid=(N // _BN,),
        in_specs=[
            pl.BlockSpec((NC, _BN, D), lambda i: (0, i, 0)),
            pl.BlockSpec((_BN, D), lambda i: (i, 0)),
            pl.BlockSpec((_BN, NW), lambda i: (i, 0)),
            pl.BlockSpec((1, D), lambda i: (0, 0)),
        ],
        out_specs=pl.BlockSpec((_BN, D), lambda i: (i, 0)),
        out_shape=jax.ShapeDtypeStruct((N, D), jnp.float32),
    )(parts, hs, degp, b.reshape(1, D))

    return out
